# baseline (speedup 1.0000x reference)
import functools

import jax
import jax.numpy as jnp
from jax import lax
from jax.experimental import pallas as pl
from jax.experimental.pallas import tpu as pltpu
from jax.experimental.pallas import tpu_sc as plsc

N_NODES = 10000
D_FEAT = 128
N_EDGES = 320000

NC = 2
NS = 16
NW = NC * NS

EDGE_B = 128
BATCH_PER_W = 80
N_BATCH = NW * BATCH_PER_W
E_PAD = N_BATCH * EDGE_B
NBUF = 2
N_ITER = BATCH_PER_W // NBUF
ACC_ROWS = N_NODES + EDGE_B

ROW_CHUNK = 200
N_CHUNK = N_NODES // ROW_CHUNK
CHUNK_PER_S = -(-N_CHUNK // NS)


def _sc_partial(x, src1d, dst1d, zrows):
    mesh = plsc.VectorSubcoreMesh(core_axis_name="c", subcore_axis_name="s")

    scratch = dict(
        acc=pltpu.VMEM_SHARED((ACC_ROWS, D_FEAT), jnp.float32),
    )
    for b in range(NBUF):
        scratch[f"rows{b}"] = pltpu.VMEM((EDGE_B, D_FEAT), jnp.float32)
        scratch[f"gsem{b}"] = pltpu.SemaphoreType.DMA
        scratch[f"ssem{b}"] = pltpu.SemaphoreType.DMA
        for p in range(2):
            scratch[f"sidx{p}{b}"] = pltpu.VMEM((EDGE_B,), jnp.int32)
            scratch[f"didx{p}{b}"] = pltpu.VMEM((EDGE_B,), jnp.int32)
            scratch[f"isem{p}{b}"] = pltpu.SemaphoreType.DMA
            scratch[f"dsem{p}{b}"] = pltpu.SemaphoreType.DMA

    @functools.partial(
        pl.kernel,
        out_type=jax.ShapeDtypeStruct((NC, N_NODES, D_FEAT), jnp.float32),
        mesh=mesh,
        scratch_types=scratch,
    )
    def kern(x_hbm, s_hbm, d_hbm, z_hbm, part_hbm, *, acc, **bufs):
        rows = [bufs[f"rows{b}"] for b in range(NBUF)]
        gsem = [bufs[f"gsem{b}"] for b in range(NBUF)]
        ssem = [bufs[f"ssem{b}"] for b in range(NBUF)]
        sidx = [[bufs[f"sidx{p}{b}"] for b in range(NBUF)] for p in range(2)]
        didx = [[bufs[f"didx{p}{b}"] for b in range(NBUF)] for p in range(2)]
        isem = [[bufs[f"isem{p}{b}"] for b in range(NBUF)] for p in range(2)]
        dsem = [[bufs[f"dsem{p}{b}"] for b in range(NBUF)] for p in range(2)]

        c = lax.axis_index("c")
        s = lax.axis_index("s")
        w = c * NS + s
        b0 = w * BATCH_PER_W

        def fire_idx(t, p):
            for b in range(NBUF):
                e0 = (b0 + t * NBUF + b) * EDGE_B
                pltpu.async_copy(s_hbm.at[pl.ds(e0, EDGE_B)], sidx[p][b], isem[p][b])
                pltpu.async_copy(d_hbm.at[pl.ds(e0, EDGE_B)], didx[p][b], dsem[p][b])

        def wait_idx(p):
            for b in range(NBUF):
                pltpu.make_async_copy(
                    s_hbm.at[pl.ds(0, EDGE_B)], sidx[p][b], isem[p][b]
                ).wait()
                pltpu.make_async_copy(
                    d_hbm.at[pl.ds(0, EDGE_B)], didx[p][b], dsem[p][b]
                ).wait()

        def scat_wait(b):
            pltpu.make_async_copy(
                x_hbm.at[pl.ds(0, EDGE_B), :], rows[b], ssem[b]
            ).wait()

        def zchunk(i, _):
            ch = s + i * NS

            @pl.when(ch < N_CHUNK)
            def _():
                pltpu.sync_copy(z_hbm, acc.at[pl.ds(ch * ROW_CHUNK, ROW_CHUNK), :])
            return 0

        lax.fori_loop(0, CHUNK_PER_S, zchunk, 0)

        @pl.when(s == 0)
        def _():
            pltpu.sync_copy(
                z_hbm.at[pl.ds(0, EDGE_B), :], acc.at[pl.ds(N_NODES, EDGE_B), :]
            )

        plsc.subcore_barrier()

        def half(u, p):
            for b in range(NBUF):
                e0 = (b0 + u * NBUF + b) * EDGE_B
                pltpu.sync_copy(s_hbm.at[pl.ds(e0, EDGE_B)], sidx[p][b])
                pltpu.sync_copy(d_hbm.at[pl.ds(e0, EDGE_B)], didx[p][b])

            gets = [
                pltpu.async_copy(x_hbm.at[sidx[p][b]], rows[b], gsem[b])
                for b in range(NBUF)
            ]
            for b in range(NBUF):
                gets[b].wait()
                pltpu.sync_copy(rows[b], acc.at[didx[p][b]], add=True)

        def outer(v, _):
            half(2 * v, 0)
            half(2 * v + 1, 1)
            return 0

        lax.fori_loop(0, N_ITER // 2, outer, 0)
        plsc.subcore_barrier()

        def wchunk(i, _):
            ch = s + i * NS

            @pl.when(ch < N_CHUNK)
            def _():
                r0 = ch * ROW_CHUNK
                pltpu.sync_copy(
                    acc.at[pl.ds(r0, ROW_CHUNK), :],
                    part_hbm.at[c, pl.ds(r0, ROW_CHUNK), :],
                )
            return 0

        lax.fori_loop(0, CHUNK_PER_S, wchunk, 0)

    return kern(x, src1d, dst1d, zrows)


def _combine(parts):
    blk = 400

    def body(p_ref, o_ref):
        o_ref[...] = p_ref[0] + p_ref[1]

    return pl.pallas_call(
        body,
        grid=(N_NODES // blk,),
        in_specs=[pl.BlockSpec((NC, blk, D_FEAT), lambda i: (0, i, 0))],
        out_specs=pl.BlockSpec((blk, D_FEAT), lambda i: (i, 0)),
        out_shape=jax.ShapeDtypeStruct((N_NODES, D_FEAT), jnp.float32),
    )(parts)


def kernel(x, edge_index):
    ei = edge_index.astype(jnp.int32)
    n_pad = E_PAD - N_EDGES
    src1d = jnp.concatenate([ei[0], jnp.zeros((n_pad,), jnp.int32)])
    pad_dst = N_NODES + (jnp.arange(n_pad, dtype=jnp.int32) % EDGE_B)
    dst1d = jnp.concatenate([ei[1], pad_dst])
    zrows = jnp.zeros((ROW_CHUNK, D_FEAT), jnp.float32)
    parts = _sc_partial(x, src1d, dst1d, zrows)
    return _combine(parts)


# speedup vs baseline: 2.3383x; 2.3383x over previous
import functools

import jax
import jax.numpy as jnp
from jax import lax
from jax.experimental import pallas as pl
from jax.experimental.pallas import tpu as pltpu
from jax.experimental.pallas import tpu_sc as plsc

N_NODES = 10000
D_FEAT = 128
N_EDGES = 320000

NC = 2
NS = 16
NW = NC * NS

EDGE_B = 128
N_BATCH = N_EDGES // EDGE_B
BATCH_PER_W = -(-N_BATCH // NW)

ROW_CHUNK = 200
N_CHUNK = N_NODES // ROW_CHUNK
CHUNK_PER_S = -(-N_CHUNK // NS)


def _sc_partial(x, edge_index):
    mesh = plsc.VectorSubcoreMesh(core_axis_name="c", subcore_axis_name="s")

    @functools.partial(
        pl.kernel,
        out_type=jax.ShapeDtypeStruct((NC, N_NODES, D_FEAT), jnp.float32),
        mesh=mesh,
        scratch_types=dict(
            zbuf=pltpu.VMEM((ROW_CHUNK, D_FEAT), jnp.float32),
            sidx=pltpu.VMEM((EDGE_B,), jnp.int32),
            didx=pltpu.VMEM((EDGE_B,), jnp.int32),
            rows=pltpu.VMEM((EDGE_B, D_FEAT), jnp.float32),
            acc=pltpu.VMEM_SHARED((N_NODES, D_FEAT), jnp.float32),
            sem=pltpu.SemaphoreType.DMA,
        ),
    )
    def kern(x_hbm, ei_hbm, part_hbm, *, zbuf, sidx, didx, rows, acc, sem):
        c = lax.axis_index("c")
        s = lax.axis_index("s")
        w = c * NS + s

        zero = jnp.zeros((16,), jnp.float32)

        def zrow(r, _):
            def zcol(k, _):
                zbuf[r, pl.ds(k * 16, 16)] = zero
                return 0
            return lax.fori_loop(0, D_FEAT // 16, zcol, 0)

        lax.fori_loop(0, ROW_CHUNK, zrow, 0)

        def zchunk(i, _):
            ch = s + i * NS

            @pl.when(ch < N_CHUNK)
            def _():
                pltpu.sync_copy(zbuf, acc.at[pl.ds(ch * ROW_CHUNK, ROW_CHUNK), :])
            return 0

        lax.fori_loop(0, CHUNK_PER_S, zchunk, 0)
        plsc.subcore_barrier()

        def ebatch(i, _):
            bid = w + i * NW

            @pl.when(bid < N_BATCH)
            def _():
                base = bid * EDGE_B
                pltpu.sync_copy(ei_hbm.at[0, pl.ds(base, EDGE_B)], sidx)
                pltpu.sync_copy(ei_hbm.at[1, pl.ds(base, EDGE_B)], didx)
                pltpu.async_copy(x_hbm.at[sidx], rows, sem).wait()
                pltpu.sync_copy(rows, acc.at[didx], add=True)
            return 0

        lax.fori_loop(0, BATCH_PER_W, ebatch, 0)
        plsc.subcore_barrier()

        def wchunk(i, _):
            ch = s + i * NS

            @pl.when(ch < N_CHUNK)
            def _():
                r0 = ch * ROW_CHUNK
                pltpu.sync_copy(
                    acc.at[pl.ds(r0, ROW_CHUNK), :],
                    part_hbm.at[c, pl.ds(r0, ROW_CHUNK), :],
                )
            return 0

        lax.fori_loop(0, CHUNK_PER_S, wchunk, 0)

    return kern(x, edge_index)


def _combine(parts):
    blk = 400

    def body(p_ref, o_ref):
        o_ref[...] = p_ref[0] + p_ref[1]

    return pl.pallas_call(
        body,
        grid=(N_NODES // blk,),
        in_specs=[pl.BlockSpec((NC, blk, D_FEAT), lambda i: (0, i, 0))],
        out_specs=pl.BlockSpec((blk, D_FEAT), lambda i: (i, 0)),
        out_shape=jax.ShapeDtypeStruct((N_NODES, D_FEAT), jnp.float32),
    )(parts)


def kernel(x, edge_index):
    ei = edge_index.astype(jnp.int32)
    parts = _sc_partial(x, ei)
    return _combine(parts)


# speedup vs baseline: 2.3404x; 1.0009x over previous
import functools

import jax
import jax.numpy as jnp
from jax import lax
from jax.experimental import pallas as pl
from jax.experimental.pallas import tpu as pltpu
from jax.experimental.pallas import tpu_sc as plsc

N_NODES = 10000
D_FEAT = 128
N_EDGES = 320000

NC = 2
NS = 16
NW = NC * NS

EDGE_B = 128
N_BATCH = N_EDGES // EDGE_B
BATCH_PER_W = -(-N_BATCH // NW)

ROW_CHUNK = 200
N_CHUNK = N_NODES // ROW_CHUNK
CHUNK_PER_S = -(-N_CHUNK // NS)


def _sc_partial(x, edge_index):
    mesh = plsc.VectorSubcoreMesh(core_axis_name="c", subcore_axis_name="s")

    @functools.partial(
        pl.kernel,
        out_type=jax.ShapeDtypeStruct((NC, N_NODES, D_FEAT), jnp.float32),
        mesh=mesh,
        scratch_types=dict(
            zbuf=pltpu.VMEM((ROW_CHUNK, D_FEAT), jnp.float32),
            sidx=pltpu.VMEM((EDGE_B,), jnp.int32),
            didx=pltpu.VMEM((EDGE_B,), jnp.int32),
            rows=pltpu.VMEM((EDGE_B, D_FEAT), jnp.float32),
            acc=pltpu.VMEM_SHARED((N_NODES, D_FEAT), jnp.float32),
            sem=pltpu.SemaphoreType.DMA,
        ),
    )
    def kern(x_hbm, ei_hbm, part_hbm, *, zbuf, sidx, didx, rows, acc, sem):
        c = lax.axis_index("c")
        s = lax.axis_index("s")
        w = c * NS + s

        zero = jnp.zeros((16,), jnp.float32)

        def zrow(r, _):
            def zcol(k, _):
                zbuf[r, pl.ds(k * 16, 16)] = zero
                return 0
            return lax.fori_loop(0, D_FEAT // 16, zcol, 0)

        lax.fori_loop(0, ROW_CHUNK, zrow, 0)

        def zchunk(i, _):
            ch = s + i * NS

            @pl.when(ch < N_CHUNK)
            def _():
                pltpu.sync_copy(zbuf, acc.at[pl.ds(ch * ROW_CHUNK, ROW_CHUNK), :])
            return 0

        lax.fori_loop(0, CHUNK_PER_S, zchunk, 0)
        plsc.subcore_barrier()

        def ebatch(i, _):
            bid = w * BATCH_PER_W + i

            @pl.when(bid < N_BATCH)
            def _():
                base = bid * EDGE_B
                pltpu.sync_copy(ei_hbm.at[0, pl.ds(base, EDGE_B)], sidx)
                pltpu.sync_copy(ei_hbm.at[1, pl.ds(base, EDGE_B)], didx)
                pltpu.async_copy(x_hbm.at[sidx], rows, sem).wait()
                pltpu.sync_copy(rows, acc.at[didx], add=True)
            return 0

        lax.fori_loop(0, BATCH_PER_W, ebatch, 0)
        plsc.subcore_barrier()

        def wchunk(i, _):
            ch = s + i * NS

            @pl.when(ch < N_CHUNK)
            def _():
                r0 = ch * ROW_CHUNK
                pltpu.sync_copy(
                    acc.at[pl.ds(r0, ROW_CHUNK), :],
                    part_hbm.at[c, pl.ds(r0, ROW_CHUNK), :],
                )
            return 0

        lax.fori_loop(0, CHUNK_PER_S, wchunk, 0)

    return kern(x, edge_index)


def _combine(parts):
    blk = 400

    def body(p_ref, o_ref):
        o_ref[...] = p_ref[0] + p_ref[1]

    return pl.pallas_call(
        body,
        grid=(N_NODES // blk,),
        in_specs=[pl.BlockSpec((NC, blk, D_FEAT), lambda i: (0, i, 0))],
        out_specs=pl.BlockSpec((blk, D_FEAT), lambda i: (i, 0)),
        out_shape=jax.ShapeDtypeStruct((N_NODES, D_FEAT), jnp.float32),
    )(parts)


def kernel(x, edge_index):
    ei = edge_index.astype(jnp.int32)
    parts = _sc_partial(x, ei)
    return _combine(parts)


# speedup vs baseline: 3.1573x; 1.3490x over previous
import functools

import jax
import jax.numpy as jnp
from jax import lax
from jax.experimental import pallas as pl
from jax.experimental.pallas import tpu as pltpu
from jax.experimental.pallas import tpu_sc as plsc

N_NODES = 10000
D_FEAT = 128
N_EDGES = 320000

NC = 2
NS = 16
NW = NC * NS

EDGE_B = 128
N_BATCH = N_EDGES // EDGE_B
BATCH_PER_W = -(-N_BATCH // NW)

ROW_CHUNK = 80
N_CHUNK = N_NODES // ROW_CHUNK
CHUNK_PER_S = -(-N_CHUNK // NS)


def _sc_partial(x, edge_index):
    mesh = plsc.VectorSubcoreMesh(core_axis_name="c", subcore_axis_name="s")

    @functools.partial(
        pl.kernel,
        out_type=jax.ShapeDtypeStruct((NC, N_NODES, D_FEAT), jnp.float32),
        mesh=mesh,
        scratch_types=dict(
            zbuf=pltpu.VMEM((ROW_CHUNK, D_FEAT), jnp.float32),
            sidx=pltpu.VMEM((EDGE_B,), jnp.int32),
            didx=pltpu.VMEM((EDGE_B,), jnp.int32),
            rows=pltpu.VMEM((EDGE_B, D_FEAT), jnp.float32),
            sidx2=pltpu.VMEM((EDGE_B,), jnp.int32),
            didx2=pltpu.VMEM((EDGE_B,), jnp.int32),
            rows2=pltpu.VMEM((EDGE_B, D_FEAT), jnp.float32),
            acc=pltpu.VMEM_SHARED((N_NODES, D_FEAT), jnp.float32),
            sem=pltpu.SemaphoreType.DMA,
            sem2=pltpu.SemaphoreType.DMA,
        ),
    )
    def kern(x_hbm, ei_hbm, part_hbm, *, zbuf, sidx, didx, rows,
             sidx2, didx2, rows2, acc, sem, sem2):
        c = lax.axis_index("c")
        s = lax.axis_index("s")
        w = c * NS + s

        zero = jnp.zeros((16,), jnp.float32)

        def zrow(r, _):
            def zcol(k, _):
                zbuf[r, pl.ds(k * 16, 16)] = zero
                return 0
            return lax.fori_loop(0, D_FEAT // 16, zcol, 0)

        lax.fori_loop(0, ROW_CHUNK, zrow, 0)

        def zchunk(i, _):
            ch = s + i * NS

            @pl.when(ch < N_CHUNK)
            def _():
                pltpu.sync_copy(zbuf, acc.at[pl.ds(ch * ROW_CHUNK, ROW_CHUNK), :])
            return 0

        lax.fori_loop(0, CHUNK_PER_S, zchunk, 0)
        plsc.subcore_barrier()

        def pair(t, _):
            bid0 = w * BATCH_PER_W + 2 * t
            bid1 = bid0 + 1
            lim = jnp.minimum((w + 1) * BATCH_PER_W, N_BATCH)

            @pl.when(bid0 < lim)
            def _():
                base = bid0 * EDGE_B
                pltpu.sync_copy(ei_hbm.at[0, pl.ds(base, EDGE_B)], sidx)
                pltpu.sync_copy(ei_hbm.at[1, pl.ds(base, EDGE_B)], didx)
                pltpu.async_copy(x_hbm.at[sidx], rows, sem)

            @pl.when(bid1 < lim)
            def _():
                base = bid1 * EDGE_B
                pltpu.sync_copy(ei_hbm.at[0, pl.ds(base, EDGE_B)], sidx2)
                pltpu.sync_copy(ei_hbm.at[1, pl.ds(base, EDGE_B)], didx2)
                pltpu.async_copy(x_hbm.at[sidx2], rows2, sem2)

            @pl.when(bid0 < lim)
            def _():
                pltpu.make_async_copy(x_hbm.at[pl.ds(0, EDGE_B), :], rows, sem).wait()
                pltpu.sync_copy(rows, acc.at[didx], add=True)

            @pl.when(bid1 < lim)
            def _():
                pltpu.make_async_copy(x_hbm.at[pl.ds(0, EDGE_B), :], rows2, sem2).wait()
                pltpu.sync_copy(rows2, acc.at[didx2], add=True)
            return 0

        lax.fori_loop(0, (BATCH_PER_W + 1) // 2, pair, 0)
        plsc.subcore_barrier()

        def wchunk(i, _):
            ch = s + i * NS

            @pl.when(ch < N_CHUNK)
            def _():
                r0 = ch * ROW_CHUNK
                pltpu.sync_copy(
                    acc.at[pl.ds(r0, ROW_CHUNK), :],
                    part_hbm.at[c, pl.ds(r0, ROW_CHUNK), :],
                )
            return 0

        lax.fori_loop(0, CHUNK_PER_S, wchunk, 0)

    return kern(x, edge_index)


def _combine(parts):
    blk = 400

    def body(p_ref, o_ref):
        o_ref[...] = p_ref[0] + p_ref[1]

    return pl.pallas_call(
        body,
        grid=(N_NODES // blk,),
        in_specs=[pl.BlockSpec((NC, blk, D_FEAT), lambda i: (0, i, 0))],
        out_specs=pl.BlockSpec((blk, D_FEAT), lambda i: (i, 0)),
        out_shape=jax.ShapeDtypeStruct((N_NODES, D_FEAT), jnp.float32),
    )(parts)


def kernel(x, edge_index):
    ei = edge_index.astype(jnp.int32)
    parts = _sc_partial(x, ei)
    return _combine(parts)


# speedup vs baseline: 3.1630x; 1.0018x over previous
import functools

import jax
import jax.numpy as jnp
from jax import lax
from jax.experimental import pallas as pl
from jax.experimental.pallas import tpu as pltpu
from jax.experimental.pallas import tpu_sc as plsc

N_NODES = 10000
D_FEAT = 128
N_EDGES = 320000

NC = 2
NS = 16
NW = NC * NS

EDGE_B = 128
N_BATCH = N_EDGES // EDGE_B
BATCH_PER_W = -(-N_BATCH // NW)

ROW_CHUNK = 80
N_CHUNK = N_NODES // ROW_CHUNK
CHUNK_PER_S = -(-N_CHUNK // NS)


def _sc_partial(x, edge_index):
    mesh = plsc.VectorSubcoreMesh(core_axis_name="c", subcore_axis_name="s")

    @functools.partial(
        pl.kernel,
        out_type=jax.ShapeDtypeStruct((NC, N_NODES, D_FEAT), jnp.float32),
        mesh=mesh,
        scratch_types=dict(
            zbuf=pltpu.VMEM((ROW_CHUNK, D_FEAT), jnp.float32),
            sidx=pltpu.VMEM((EDGE_B,), jnp.int32),
            didx=pltpu.VMEM((EDGE_B,), jnp.int32),
            rows=pltpu.VMEM((EDGE_B, D_FEAT), jnp.float32),
            sidx2=pltpu.VMEM((EDGE_B,), jnp.int32),
            didx2=pltpu.VMEM((EDGE_B,), jnp.int32),
            rows2=pltpu.VMEM((EDGE_B, D_FEAT), jnp.float32),
            acc=pltpu.VMEM_SHARED((N_NODES, D_FEAT), jnp.float32),
            sem=pltpu.SemaphoreType.DMA,
            sem2=pltpu.SemaphoreType.DMA,
            ssem=pltpu.SemaphoreType.DMA,
            ssem2=pltpu.SemaphoreType.DMA,
        ),
    )
    def kern(x_hbm, ei_hbm, part_hbm, *, zbuf, sidx, didx, rows,
             sidx2, didx2, rows2, acc, sem, sem2, ssem, ssem2):
        c = lax.axis_index("c")
        s = lax.axis_index("s")
        w = c * NS + s

        zero = jnp.zeros((16,), jnp.float32)

        def zrow(r, _):
            def zcol(k, _):
                zbuf[r, pl.ds(k * 16, 16)] = zero
                return 0
            return lax.fori_loop(0, D_FEAT // 16, zcol, 0)

        lax.fori_loop(0, ROW_CHUNK, zrow, 0)

        def zchunk(i, _):
            ch = s + i * NS

            @pl.when(ch < N_CHUNK)
            def _():
                pltpu.sync_copy(zbuf, acc.at[pl.ds(ch * ROW_CHUNK, ROW_CHUNK), :])
            return 0

        lax.fori_loop(0, CHUNK_PER_S, zchunk, 0)
        plsc.subcore_barrier()

        def pair(t, _):
            bid0 = w * BATCH_PER_W + 2 * t
            bid1 = bid0 + 1
            lim = jnp.minimum((w + 1) * BATCH_PER_W, N_BATCH)

            @pl.when((t > 0) & (bid0 - 2 < lim))
            def _():
                pltpu.make_async_copy(x_hbm.at[pl.ds(0, EDGE_B), :], rows, ssem).wait()

            @pl.when((t > 0) & (bid1 - 2 < lim))
            def _():
                pltpu.make_async_copy(x_hbm.at[pl.ds(0, EDGE_B), :], rows2, ssem2).wait()

            @pl.when(bid0 < lim)
            def _():
                base = bid0 * EDGE_B
                pltpu.sync_copy(ei_hbm.at[0, pl.ds(base, EDGE_B)], sidx)
                pltpu.sync_copy(ei_hbm.at[1, pl.ds(base, EDGE_B)], didx)
                pltpu.async_copy(x_hbm.at[sidx], rows, sem)

            @pl.when(bid1 < lim)
            def _():
                base = bid1 * EDGE_B
                pltpu.sync_copy(ei_hbm.at[0, pl.ds(base, EDGE_B)], sidx2)
                pltpu.sync_copy(ei_hbm.at[1, pl.ds(base, EDGE_B)], didx2)
                pltpu.async_copy(x_hbm.at[sidx2], rows2, sem2)

            @pl.when(bid0 < lim)
            def _():
                pltpu.make_async_copy(x_hbm.at[pl.ds(0, EDGE_B), :], rows, sem).wait()
                pltpu.async_copy(rows, acc.at[didx], ssem, add=True)

            @pl.when(bid1 < lim)
            def _():
                pltpu.make_async_copy(x_hbm.at[pl.ds(0, EDGE_B), :], rows2, sem2).wait()
                pltpu.async_copy(rows2, acc.at[didx2], ssem2, add=True)
            return 0

        n_pair = (BATCH_PER_W + 1) // 2
        lax.fori_loop(0, n_pair, pair, 0)
        last0 = w * BATCH_PER_W + 2 * (n_pair - 1)
        lim_f = jnp.minimum((w + 1) * BATCH_PER_W, N_BATCH)

        @pl.when(last0 < lim_f)
        def _():
            pltpu.make_async_copy(x_hbm.at[pl.ds(0, EDGE_B), :], rows, ssem).wait()

        @pl.when(last0 + 1 < lim_f)
        def _():
            pltpu.make_async_copy(x_hbm.at[pl.ds(0, EDGE_B), :], rows2, ssem2).wait()
        plsc.subcore_barrier()

        def wchunk(i, _):
            ch = s + i * NS

            @pl.when(ch < N_CHUNK)
            def _():
                r0 = ch * ROW_CHUNK
                pltpu.sync_copy(
                    acc.at[pl.ds(r0, ROW_CHUNK), :],
                    part_hbm.at[c, pl.ds(r0, ROW_CHUNK), :],
                )
            return 0

        lax.fori_loop(0, CHUNK_PER_S, wchunk, 0)

    return kern(x, edge_index)


def _combine(parts):
    blk = 400

    def body(p_ref, o_ref):
        o_ref[...] = p_ref[0] + p_ref[1]

    return pl.pallas_call(
        body,
        grid=(N_NODES // blk,),
        in_specs=[pl.BlockSpec((NC, blk, D_FEAT), lambda i: (0, i, 0))],
        out_specs=pl.BlockSpec((blk, D_FEAT), lambda i: (i, 0)),
        out_shape=jax.ShapeDtypeStruct((N_NODES, D_FEAT), jnp.float32),
    )(parts)


def kernel(x, edge_index):
    ei = edge_index.astype(jnp.int32)
    parts = _sc_partial(x, ei)
    return _combine(parts)


# speedup vs baseline: 3.6437x; 1.1520x over previous
import functools

import jax
import jax.numpy as jnp
from jax import lax
from jax.experimental import pallas as pl
from jax.experimental.pallas import tpu as pltpu
from jax.experimental.pallas import tpu_sc as plsc

N_NODES = 10000
D_FEAT = 128
N_EDGES = 320000

NC = 2
NS = 16
NW = NC * NS

EDGE_B = 128
N_BATCH = N_EDGES // EDGE_B
BATCH_PER_W = -(-N_BATCH // NW)

ROW_CHUNK = 80
N_CHUNK = N_NODES // ROW_CHUNK
CHUNK_PER_S = -(-N_CHUNK // NS)


def _sc_partial(x, edge_index):
    mesh = plsc.VectorSubcoreMesh(core_axis_name="c", subcore_axis_name="s")

    scratch = dict(
        zbuf=pltpu.VMEM((ROW_CHUNK, D_FEAT), jnp.float32),
        rows=pltpu.VMEM((EDGE_B, D_FEAT), jnp.float32),
        rows2=pltpu.VMEM((EDGE_B, D_FEAT), jnp.float32),
        acc=pltpu.VMEM_SHARED((N_NODES, D_FEAT), jnp.float32),
        gsem=pltpu.SemaphoreType.DMA,
        gsem2=pltpu.SemaphoreType.DMA,
        ssem=pltpu.SemaphoreType.DMA,
        ssem2=pltpu.SemaphoreType.DMA,
    )
    for par in "AB":
        for b in range(2):
            scratch[f"sidx{b}{par}"] = pltpu.VMEM((EDGE_B,), jnp.int32)
            scratch[f"didx{b}{par}"] = pltpu.VMEM((EDGE_B,), jnp.int32)
            scratch[f"issem{b}{par}"] = pltpu.SemaphoreType.DMA
            scratch[f"idsem{b}{par}"] = pltpu.SemaphoreType.DMA

    @functools.partial(
        pl.kernel,
        out_type=jax.ShapeDtypeStruct((NC, N_NODES, D_FEAT), jnp.float32),
        mesh=mesh,
        scratch_types=scratch,
    )
    def kern(x_hbm, ei_hbm, part_hbm, *, zbuf, rows, rows2, acc,
             gsem, gsem2, ssem, ssem2, **ibufs):
        c = lax.axis_index("c")
        s = lax.axis_index("s")
        w = c * NS + s
        lim = jnp.minimum((w + 1) * BATCH_PER_W, N_BATCH)
        rowbuf = [rows, rows2]
        rsem = [gsem, gsem2]
        wsem = [ssem, ssem2]

        def idx_fire(t, par):
            for b in range(2):
                bid = w * BATCH_PER_W + 2 * t + b

                @pl.when(bid < lim)
                def _():
                    base = bid * EDGE_B
                    pltpu.async_copy(ei_hbm.at[0, pl.ds(base, EDGE_B)],
                                     ibufs[f"sidx{b}{par}"],
                                     ibufs[f"issem{b}{par}"])
                    pltpu.async_copy(ei_hbm.at[1, pl.ds(base, EDGE_B)],
                                     ibufs[f"didx{b}{par}"],
                                     ibufs[f"idsem{b}{par}"])

        def idx_wait(t, par):
            for b in range(2):
                bid = w * BATCH_PER_W + 2 * t + b

                @pl.when(bid < lim)
                def _():
                    pltpu.make_async_copy(ei_hbm.at[0, pl.ds(0, EDGE_B)],
                                          ibufs[f"sidx{b}{par}"],
                                          ibufs[f"issem{b}{par}"]).wait()
                    pltpu.make_async_copy(ei_hbm.at[1, pl.ds(0, EDGE_B)],
                                          ibufs[f"didx{b}{par}"],
                                          ibufs[f"idsem{b}{par}"]).wait()

        def scat_drain(t):
            for b in range(2):
                bid = w * BATCH_PER_W + 2 * t + b

                @pl.when((t >= 0) & (bid < lim))
                def _():
                    pltpu.make_async_copy(x_hbm.at[pl.ds(0, EDGE_B), :],
                                          rowbuf[b], wsem[b]).wait()

        idx_fire(0, "A")

        zero = jnp.zeros((16,), jnp.float32)

        def zrow(r, _):
            def zcol(k, _):
                zbuf[r, pl.ds(k * 16, 16)] = zero
                return 0
            return lax.fori_loop(0, D_FEAT // 16, zcol, 0)

        lax.fori_loop(0, ROW_CHUNK, zrow, 0)

        def zchunk(i, _):
            ch = s + i * NS

            @pl.when(ch < N_CHUNK)
            def _():
                pltpu.sync_copy(zbuf, acc.at[pl.ds(ch * ROW_CHUNK, ROW_CHUNK), :])
            return 0

        lax.fori_loop(0, CHUNK_PER_S, zchunk, 0)
        plsc.subcore_barrier()

        def pair(t, par, nxt):
            scat_drain(t - 1)
            idx_wait(t, par)
            idx_fire(t + 1, nxt)
            gets = []
            for b in range(2):
                bid = w * BATCH_PER_W + 2 * t + b

                @pl.when(bid < lim)
                def _():
                    pltpu.async_copy(x_hbm.at[ibufs[f"sidx{b}{par}"]],
                                     rowbuf[b], rsem[b])
            for b in range(2):
                bid = w * BATCH_PER_W + 2 * t + b

                @pl.when(bid < lim)
                def _():
                    pltpu.make_async_copy(x_hbm.at[pl.ds(0, EDGE_B), :],
                                          rowbuf[b], rsem[b]).wait()
                    pltpu.async_copy(rowbuf[b], acc.at[ibufs[f"didx{b}{par}"]],
                                     wsem[b], add=True)

        n_pair = (BATCH_PER_W + 1) // 2

        def quad(q, _):
            pair(2 * q, "A", "B")
            pair(2 * q + 1, "B", "A")
            return 0

        lax.fori_loop(0, n_pair // 2, quad, 0)
        scat_drain(n_pair - 1)
        plsc.subcore_barrier()

        def wchunk(i, _):
            ch = s + i * NS

            @pl.when(ch < N_CHUNK)
            def _():
                r0 = ch * ROW_CHUNK
                pltpu.sync_copy(
                    acc.at[pl.ds(r0, ROW_CHUNK), :],
                    part_hbm.at[c, pl.ds(r0, ROW_CHUNK), :],
                )
            return 0

        lax.fori_loop(0, CHUNK_PER_S, wchunk, 0)

    return kern(x, edge_index)


def _combine(parts):
    blk = 400

    def body(p_ref, o_ref):
        o_ref[...] = p_ref[0] + p_ref[1]

    return pl.pallas_call(
        body,
        grid=(N_NODES // blk,),
        in_specs=[pl.BlockSpec((NC, blk, D_FEAT), lambda i: (0, i, 0))],
        out_specs=pl.BlockSpec((blk, D_FEAT), lambda i: (i, 0)),
        out_shape=jax.ShapeDtypeStruct((N_NODES, D_FEAT), jnp.float32),
    )(parts)


def kernel(x, edge_index):
    ei = edge_index.astype(jnp.int32)
    parts = _sc_partial(x, ei)
    return _combine(parts)


# speedup vs baseline: 3.6448x; 1.0003x over previous
import functools

import jax
import jax.numpy as jnp
from jax import lax
from jax.experimental import pallas as pl
from jax.experimental.pallas import tpu as pltpu
from jax.experimental.pallas import tpu_sc as plsc

N_NODES = 10000
D_FEAT = 128
N_EDGES = 320000

NC = 2
NS = 16
NW = NC * NS

EDGE_B = 128
N_BATCH = N_EDGES // EDGE_B
BATCH_PER_W = -(-N_BATCH // NW)

ROW_CHUNK = 80
N_CHUNK = N_NODES // ROW_CHUNK
CHUNK_PER_S = -(-N_CHUNK // NS)


def _sc_partial(x, edge_index):
    mesh = plsc.VectorSubcoreMesh(core_axis_name="c", subcore_axis_name="s")

    scratch = dict(
        rows=pltpu.VMEM((EDGE_B, D_FEAT), jnp.float32),
        rows2=pltpu.VMEM((EDGE_B, D_FEAT), jnp.float32),
        acc=pltpu.VMEM_SHARED((N_NODES, D_FEAT), jnp.float32),
        gsem=pltpu.SemaphoreType.DMA,
        gsem2=pltpu.SemaphoreType.DMA,
        ssem=pltpu.SemaphoreType.DMA,
        ssem2=pltpu.SemaphoreType.DMA,
    )
    for par in "AB":
        for b in range(2):
            scratch[f"sidx{b}{par}"] = pltpu.VMEM((EDGE_B,), jnp.int32)
            scratch[f"didx{b}{par}"] = pltpu.VMEM((EDGE_B,), jnp.int32)
            scratch[f"issem{b}{par}"] = pltpu.SemaphoreType.DMA
            scratch[f"idsem{b}{par}"] = pltpu.SemaphoreType.DMA

    @functools.partial(
        pl.kernel,
        out_type=jax.ShapeDtypeStruct((NC, N_NODES, D_FEAT), jnp.float32),
        mesh=mesh,
        scratch_types=scratch,
    )
    def kern(x_hbm, ei_hbm, part_hbm, *, rows, rows2, acc,
             gsem, gsem2, ssem, ssem2, **ibufs):
        c = lax.axis_index("c")
        s = lax.axis_index("s")
        w = c * NS + s
        lim = jnp.minimum((w + 1) * BATCH_PER_W, N_BATCH)
        rowbuf = [rows, rows2]
        rsem = [gsem, gsem2]
        wsem = [ssem, ssem2]

        def idx_fire(t, par):
            for b in range(2):
                bid = w * BATCH_PER_W + 2 * t + b

                @pl.when(bid < lim)
                def _():
                    base = bid * EDGE_B
                    pltpu.async_copy(ei_hbm.at[0, pl.ds(base, EDGE_B)],
                                     ibufs[f"sidx{b}{par}"],
                                     ibufs[f"issem{b}{par}"])
                    pltpu.async_copy(ei_hbm.at[1, pl.ds(base, EDGE_B)],
                                     ibufs[f"didx{b}{par}"],
                                     ibufs[f"idsem{b}{par}"])

        def idx_wait(t, par):
            for b in range(2):
                bid = w * BATCH_PER_W + 2 * t + b

                @pl.when(bid < lim)
                def _():
                    pltpu.make_async_copy(ei_hbm.at[0, pl.ds(0, EDGE_B)],
                                          ibufs[f"sidx{b}{par}"],
                                          ibufs[f"issem{b}{par}"]).wait()
                    pltpu.make_async_copy(ei_hbm.at[1, pl.ds(0, EDGE_B)],
                                          ibufs[f"didx{b}{par}"],
                                          ibufs[f"idsem{b}{par}"]).wait()

        def scat_drain(t):
            for b in range(2):
                bid = w * BATCH_PER_W + 2 * t + b

                @pl.when((t >= 0) & (bid < lim))
                def _():
                    pltpu.make_async_copy(x_hbm.at[pl.ds(0, EDGE_B), :],
                                          rowbuf[b], wsem[b]).wait()

        idx_fire(0, "A")

        zero = jnp.zeros((16,), jnp.float32)

        def zrow(r, _):
            def zcol(k, _):
                rows[r, pl.ds(k * 16, 16)] = zero
                return 0
            return lax.fori_loop(0, D_FEAT // 16, zcol, 0)

        lax.fori_loop(0, ROW_CHUNK, zrow, 0)

        def zchunk(i, _):
            ch = s + i * NS

            @pl.when(ch < N_CHUNK)
            def _():
                pltpu.async_copy(rows.at[pl.ds(0, ROW_CHUNK), :],
                                 acc.at[pl.ds(ch * ROW_CHUNK, ROW_CHUNK), :],
                                 ssem)
            return 0

        lax.fori_loop(0, CHUNK_PER_S, zchunk, 0)

        def zdrain(i, _):
            ch = s + i * NS

            @pl.when(ch < N_CHUNK)
            def _():
                pltpu.make_async_copy(rows.at[pl.ds(0, ROW_CHUNK), :],
                                      acc.at[pl.ds(0, ROW_CHUNK), :],
                                      ssem).wait()
            return 0

        lax.fori_loop(0, CHUNK_PER_S, zdrain, 0)
        plsc.subcore_barrier()

        def pair(t, par, nxt):
            scat_drain(t - 1)
            idx_wait(t, par)
            idx_fire(t + 1, nxt)
            gets = []
            for b in range(2):
                bid = w * BATCH_PER_W + 2 * t + b

                @pl.when(bid < lim)
                def _():
                    pltpu.async_copy(x_hbm.at[ibufs[f"sidx{b}{par}"]],
                                     rowbuf[b], rsem[b])
            for b in range(2):
                bid = w * BATCH_PER_W + 2 * t + b

                @pl.when(bid < lim)
                def _():
                    pltpu.make_async_copy(x_hbm.at[pl.ds(0, EDGE_B), :],
                                          rowbuf[b], rsem[b]).wait()
                    pltpu.async_copy(rowbuf[b], acc.at[ibufs[f"didx{b}{par}"]],
                                     wsem[b], add=True)

        n_pair = (BATCH_PER_W + 1) // 2

        def quad(q, _):
            pair(2 * q, "A", "B")
            pair(2 * q + 1, "B", "A")
            return 0

        lax.fori_loop(0, n_pair // 2, quad, 0)
        scat_drain(n_pair - 1)
        plsc.subcore_barrier()

        def wchunk(i, _):
            ch = s + i * NS

            @pl.when(ch < N_CHUNK)
            def _():
                r0 = ch * ROW_CHUNK
                pltpu.async_copy(
                    acc.at[pl.ds(r0, ROW_CHUNK), :],
                    part_hbm.at[c, pl.ds(r0, ROW_CHUNK), :],
                    ssem,
                )
            return 0

        lax.fori_loop(0, CHUNK_PER_S, wchunk, 0)

        def wdrain(i, _):
            ch = s + i * NS

            @pl.when(ch < N_CHUNK)
            def _():
                pltpu.make_async_copy(
                    acc.at[pl.ds(0, ROW_CHUNK), :],
                    part_hbm.at[c, pl.ds(0, ROW_CHUNK), :],
                    ssem,
                ).wait()
            return 0

        lax.fori_loop(0, CHUNK_PER_S, wdrain, 0)

    return kern(x, edge_index)


def _combine(parts):
    blk = 400

    def body(p_ref, o_ref):
        o_ref[...] = p_ref[0] + p_ref[1]

    return pl.pallas_call(
        body,
        grid=(N_NODES // blk,),
        in_specs=[pl.BlockSpec((NC, blk, D_FEAT), lambda i: (0, i, 0))],
        out_specs=pl.BlockSpec((blk, D_FEAT), lambda i: (i, 0)),
        out_shape=jax.ShapeDtypeStruct((N_NODES, D_FEAT), jnp.float32),
    )(parts)


def kernel(x, edge_index):
    ei = edge_index.astype(jnp.int32)
    parts = _sc_partial(x, ei)
    return _combine(parts)
